# Initial kernel scaffold; baseline (speedup 1.0000x reference)
import jax
import jax.numpy as jnp
from jax.experimental import pallas as pl


def kernel(hidden_states, p_mask, cls_index, W_start, b_start, W_e0, b_e0, ln_g, ln_b, W_e1, b_e1, W_a0, b_a0, W_a1):
    raise NotImplementedError("write your pallas kernel here")


# speedup vs baseline: 1.4733x; 1.4733x over previous
import jax
import jax.numpy as jnp
from jax.experimental import pallas as pl
from jax.experimental.pallas import tpu as pltpu

_EPS = 1e-12
_NEG = -1e30


def _bdot(a, b):
    return jax.lax.dot_general(a, b, (((1,), (0,)), ((), ())),
                               preferred_element_type=jnp.float32)


def _topk_rows(scores, iota, kk, kio, s_len):
    vvec = jnp.zeros((1, kk), jnp.float32)
    ivec = jnp.zeros((1, kk), jnp.int32)
    cur = scores
    for k in range(kk):
        mk = jnp.max(cur)
        ik = jnp.min(jnp.where(cur == mk, iota, s_len))
        vvec = jnp.where(kio == k, mk, vvec)
        ivec = jnp.where(kio == k, ik, ivec)
        cur = jnp.where(iota == ik, _NEG, cur)
    return vvec, ivec


def _start_body(cls_ref, hs_ref, pm_ref, wst_ref, bst_ref, we0b_ref, be0_ref,
                wa0t_ref, wa0b_ref, ba0_ref, wa1_ref,
                stv_ref, sti_ref, off_ref, clsl_ref):
    b = pl.program_id(0)
    S, H = hs_ref.shape[1], hs_ref.shape[2]
    K1 = stv_ref.shape[2]
    hs = hs_ref[0]
    pm = pm_ref[0]
    iota = jax.lax.broadcasted_iota(jnp.int32, (S, 1), 0)
    kio = jax.lax.broadcasted_iota(jnp.int32, (1, K1), 1)

    sl = _bdot(hs, wst_ref[...])
    sl = (sl + bst_ref[0, 0]) * (1.0 - pm) - 1e30 * pm
    m0 = jnp.max(sl)
    e0 = jnp.exp(sl - m0)
    d0 = jnp.sum(e0)

    lvec, ivec = _topk_rows(sl, iota, K1, kio, S)
    stv_ref[0] = jnp.exp(lvec - m0) / d0
    sti_ref[0] = ivec

    io8 = jax.lax.broadcasted_iota(jnp.int32, (8, 1), 0)

    def _gather_row(ix):
        ia = (ix // 8) * 8
        blk = hs_ref[0, pl.ds(ia, 8), :].astype(jnp.float32)
        sel = jnp.where(io8 == ix - ia, blk, 0.0)
        return jnp.sum(sel, axis=0, keepdims=True)

    rows = []
    for k in range(K1):
        ik = jnp.min(jnp.where(kio == k, ivec, S))
        rows.append(_gather_row(ik))
    ss = jnp.concatenate(rows, axis=0).astype(jnp.bfloat16)
    off_ref[0] = _bdot(ss, we0b_ref[...]) + be0_ref[...]

    agg = jax.lax.dot_general(e0.astype(jnp.bfloat16), hs,
                              (((0,), (0,)), ((), ())),
                              preferred_element_type=jnp.float32)
    agg = agg * (1.0 / d0)
    cb = cls_ref[b]
    ctok = _gather_row(cb).astype(jnp.bfloat16)
    xa = jnp.tanh(_bdot(agg.astype(jnp.bfloat16), wa0t_ref[...])
                  + _bdot(ctok, wa0b_ref[...]) + ba0_ref[...])
    clsl_ref[0] = _bdot(xa.astype(jnp.bfloat16), wa1_ref[...])


def _end_logits_body(hs_ref, pm_ref, off_ref, we0t_ref, lng_ref, lnb_ref,
                     we1_ref, be1_ref, el_ref):
    T, H = hs_ref.shape[1], hs_ref.shape[2]
    K1 = off_ref.shape[1]
    hs = hs_ref[0]
    pm = pm_ref[0]
    keep = 1.0 - pm

    base = _bdot(hs, we0t_ref[...])
    lng = lng_ref[...]
    lnb = lnb_ref[...]
    inv_h = 1.0 / H

    cols = []
    for k in range(K1):
        xk = jnp.tanh(base + off_ref[0, k:k + 1, :])
        mu = jnp.sum(xk, axis=1, keepdims=True) * inv_h
        xc = xk - mu
        var = jnp.sum(xc * xc, axis=1, keepdims=True) * inv_h
        xn = xc / jnp.sqrt(var + _EPS) * lng + lnb
        el = _bdot(xn.astype(jnp.bfloat16), we1_ref[...]) + be1_ref[0, 0]
        cols.append(el * keep - 1e30 * pm)
    el_ref[0] = jnp.concatenate(cols, axis=1)


def _end_topk_body(el_ref, etv_ref, eti_ref):
    S = el_ref.shape[1]
    K1 = etv_ref.shape[1]
    K2 = etv_ref.shape[2]
    iota = jax.lax.broadcasted_iota(jnp.int32, (S, 1), 0)
    kio = jax.lax.broadcasted_iota(jnp.int32, (1, K2), 1)
    el = el_ref[0]
    vrows = []
    irows = []
    for k in range(K1):
        col = el[:, k:k + 1]
        me = jnp.max(col)
        de = jnp.sum(jnp.exp(col - me))
        lvec, ivec = _topk_rows(col, iota, K2, kio, S)
        vrows.append(jnp.exp(lvec - me) / de)
        irows.append(ivec)
    etv_ref[0] = jnp.concatenate(vrows, axis=0)
    eti_ref[0] = jnp.concatenate(irows, axis=0)


def kernel(hidden_states, p_mask, cls_index, W_start, b_start, W_e0, b_e0,
           ln_g, ln_b, W_e1, b_e1, W_a0, b_a0, W_a1):
    B, S, H = hidden_states.shape
    K1, K2 = 5, 5
    TILE = 1024
    NT = S // TILE

    bf = jnp.bfloat16
    hs_bf = hidden_states.astype(bf)
    cls_i = cls_index.astype(jnp.int32)
    pm3 = p_mask[:, :, None]

    full = lambda shape: pl.BlockSpec(shape, lambda *a: (0,) * len(shape))

    stv, sti, off, clsl = pl.pallas_call(
        _start_body,
        grid_spec=pltpu.PrefetchScalarGridSpec(
            num_scalar_prefetch=1,
            grid=(B,),
            in_specs=[
                pl.BlockSpec((1, S, H), lambda b, c: (b, 0, 0)),
                pl.BlockSpec((1, S, 1), lambda b, c: (b, 0, 0)),
                full((H, 1)),
                full((1, 1)),
                full((H, H)),
                full((1, H)),
                full((H, H)),
                full((H, H)),
                full((1, H)),
                full((H, 1)),
            ],
            out_specs=[
                pl.BlockSpec((1, 1, K1), lambda b, c: (b, 0, 0)),
                pl.BlockSpec((1, 1, K1), lambda b, c: (b, 0, 0)),
                pl.BlockSpec((1, K1, H), lambda b, c: (b, 0, 0)),
                pl.BlockSpec((1, 1, 1), lambda b, c: (b, 0, 0)),
            ],
        ),
        out_shape=[
            jax.ShapeDtypeStruct((B, 1, K1), jnp.float32),
            jax.ShapeDtypeStruct((B, 1, K1), jnp.int32),
            jax.ShapeDtypeStruct((B, K1, H), jnp.float32),
            jax.ShapeDtypeStruct((B, 1, 1), jnp.float32),
        ],
        compiler_params=pltpu.CompilerParams(
            dimension_semantics=("arbitrary",),
        ),
    )(cls_i, hs_bf, pm3, W_start.astype(bf), b_start.reshape(1, 1),
      W_e0[H:].astype(bf), b_e0.reshape(1, H), W_a0[:H].astype(bf),
      W_a0[H:].astype(bf), b_a0.reshape(1, H), W_a1.astype(bf))

    el = pl.pallas_call(
        _end_logits_body,
        grid=(B, NT),
        in_specs=[
            pl.BlockSpec((1, TILE, H), lambda b, t: (b, t, 0)),
            pl.BlockSpec((1, TILE, 1), lambda b, t: (b, t, 0)),
            pl.BlockSpec((1, K1, H), lambda b, t: (b, 0, 0)),
            full((H, H)),
            full((1, H)),
            full((1, H)),
            full((H, 1)),
            full((1, 1)),
        ],
        out_specs=pl.BlockSpec((1, TILE, K1), lambda b, t: (b, t, 0)),
        out_shape=jax.ShapeDtypeStruct((B, S, K1), jnp.float32),
        compiler_params=pltpu.CompilerParams(
            dimension_semantics=("parallel", "arbitrary"),
        ),
    )(hs_bf, pm3, off, W_e0[:H].astype(bf), ln_g.reshape(1, H),
      ln_b.reshape(1, H), W_e1.astype(bf), b_e1.reshape(1, 1))

    etv, eti = pl.pallas_call(
        _end_topk_body,
        grid=(B,),
        in_specs=[pl.BlockSpec((1, S, K1), lambda b: (b, 0, 0))],
        out_specs=[
            pl.BlockSpec((1, K1, K2), lambda b: (b, 0, 0)),
            pl.BlockSpec((1, K1, K2), lambda b: (b, 0, 0)),
        ],
        out_shape=[
            jax.ShapeDtypeStruct((B, K1, K2), jnp.float32),
            jax.ShapeDtypeStruct((B, K1, K2), jnp.int32),
        ],
        compiler_params=pltpu.CompilerParams(
            dimension_semantics=("arbitrary",),
        ),
    )(el)

    start_top_log_probs = stv[:, 0, :]
    start_top_index = sti[:, 0, :]
    end_top_log_probs = jnp.transpose(etv, (0, 2, 1)).reshape(B, K1 * K2)
    end_top_index = jnp.transpose(eti, (0, 2, 1)).reshape(B, K1 * K2)
    cls_logits = clsl.reshape(B)
    return (start_top_log_probs, start_top_index, end_top_log_probs,
            end_top_index, cls_logits)


# speedup vs baseline: 2.2956x; 1.5581x over previous
import jax
import jax.numpy as jnp
from jax.experimental import pallas as pl
from jax.experimental.pallas import tpu as pltpu

_EPS = 1e-12
_NEG = -1e30


def _bdot(a, b):
    return jax.lax.dot_general(a, b, (((1,), (0,)), ((), ())),
                               preferred_element_type=jnp.float32)


def _topk_rows(scores, iota, kk, kio, s_len):
    vvec = jnp.zeros((1, kk), jnp.float32)
    ivec = jnp.zeros((1, kk), jnp.int32)
    cur = scores
    for k in range(kk):
        mk = jnp.max(cur)
        ik = jnp.min(jnp.where(cur == mk, iota, s_len))
        vvec = jnp.where(kio == k, mk, vvec)
        ivec = jnp.where(kio == k, ik, ivec)
        cur = jnp.where(iota == ik, _NEG, cur)
    return vvec, ivec


def _start_body(cls_ref, hs_ref, pm_ref, wst_ref, bst_ref, we0b_ref, be0_ref,
                wa0t_ref, wa0b_ref, ba0_ref, wa1_ref,
                stv_ref, sti_ref, off_ref, clsl_ref):
    b = pl.program_id(0)
    S, H = hs_ref.shape[1], hs_ref.shape[2]
    K1 = stv_ref.shape[2]
    hs = hs_ref[0]
    pm = pm_ref[0]
    iota = jax.lax.broadcasted_iota(jnp.int32, (1, S), 1)
    kio = jax.lax.broadcasted_iota(jnp.int32, (1, K1), 1)

    sl = jax.lax.dot_general(wst_ref[...], hs, (((1,), (1,)), ((), ())),
                             preferred_element_type=jnp.float32)
    sl = (sl + bst_ref[0, 0]) * (1.0 - pm) - 1e30 * pm
    m0 = jnp.max(sl)
    e0 = jnp.exp(sl - m0)
    d0 = jnp.sum(e0)

    lvec, ivec = _topk_rows(sl, iota, K1, kio, S)
    stv_ref[0] = jnp.exp(lvec - m0) / d0
    sti_ref[0] = ivec

    io8 = jax.lax.broadcasted_iota(jnp.int32, (8, 1), 0)

    def _gather_row(ix):
        ia = (ix // 8) * 8
        blk = hs_ref[0, pl.ds(ia, 8), :].astype(jnp.float32)
        sel = jnp.where(io8 == ix - ia, blk, 0.0)
        return jnp.sum(sel, axis=0, keepdims=True)

    rows = []
    for k in range(K1):
        ik = jnp.min(jnp.where(kio == k, ivec, S))
        rows.append(_gather_row(ik))
    ss = jnp.concatenate(rows, axis=0).astype(jnp.bfloat16)
    off_ref[0] = _bdot(ss, we0b_ref[...]) + be0_ref[...]

    agg = _bdot(e0.astype(jnp.bfloat16), hs)
    agg = agg * (1.0 / d0)
    cb = cls_ref[b]
    ctok = _gather_row(cb).astype(jnp.bfloat16)
    xa = jnp.tanh(_bdot(agg.astype(jnp.bfloat16), wa0t_ref[...])
                  + _bdot(ctok, wa0b_ref[...]) + ba0_ref[...])
    clsl_ref[0] = _bdot(xa.astype(jnp.bfloat16), wa1_ref[...])


def _end_logits_body(hs_ref, pm_ref, off_ref, we0t_ref, lng_ref, lnb_ref,
                     we1_ref, be1_ref, el_ref):
    T, H = hs_ref.shape[1], hs_ref.shape[2]
    K1 = off_ref.shape[1]
    hs = hs_ref[0]
    pm = pm_ref[0]
    keep = 1.0 - pm

    base = _bdot(hs, we0t_ref[...])
    lng = lng_ref[...]
    lnb = lnb_ref[...]
    inv_h = 1.0 / H

    cols = []
    for k in range(K1):
        xk = jnp.tanh(base + off_ref[0, k:k + 1, :])
        mu = jnp.sum(xk, axis=1, keepdims=True) * inv_h
        xc = xk - mu
        var = jnp.sum(xc * xc, axis=1, keepdims=True) * inv_h
        xn = xc / jnp.sqrt(var + _EPS) * lng + lnb
        el = _bdot(xn.astype(jnp.bfloat16), we1_ref[...]) + be1_ref[0, 0]
        cols.append(el * keep - 1e30 * pm)
    el_ref[0] = jnp.concatenate(cols, axis=1)


def _end_topk_body(el_ref, etv_ref, eti_ref):
    S = el_ref.shape[2]
    K1 = etv_ref.shape[1]
    K2 = etv_ref.shape[2]
    iota = jax.lax.broadcasted_iota(jnp.int32, (1, S), 1)
    kio = jax.lax.broadcasted_iota(jnp.int32, (1, K2), 1)
    el = el_ref[0]
    vrows = []
    irows = []
    for k in range(K1):
        row = el[k:k + 1, :]
        me = jnp.max(row)
        de = jnp.sum(jnp.exp(row - me))
        lvec, ivec = _topk_rows(row, iota, K2, kio, S)
        vrows.append(jnp.exp(lvec - me) / de)
        irows.append(ivec)
    etv_ref[0] = jnp.concatenate(vrows, axis=0)
    eti_ref[0] = jnp.concatenate(irows, axis=0)


def kernel(hidden_states, p_mask, cls_index, W_start, b_start, W_e0, b_e0,
           ln_g, ln_b, W_e1, b_e1, W_a0, b_a0, W_a1):
    B, S, H = hidden_states.shape
    K1, K2 = 5, 5
    TILE = 1024
    NT = S // TILE

    bf = jnp.bfloat16
    hs_bf = hidden_states.astype(bf)
    cls_i = cls_index.astype(jnp.int32)
    pm3 = p_mask[:, :, None]
    pmr = p_mask[:, None, :]

    full = lambda shape: pl.BlockSpec(shape, lambda *a: (0,) * len(shape))

    stv, sti, off, clsl = pl.pallas_call(
        _start_body,
        grid_spec=pltpu.PrefetchScalarGridSpec(
            num_scalar_prefetch=1,
            grid=(B,),
            in_specs=[
                pl.BlockSpec((1, S, H), lambda b, c: (b, 0, 0)),
                pl.BlockSpec((1, 1, S), lambda b, c: (b, 0, 0)),
                full((1, H)),
                full((1, 1)),
                full((H, H)),
                full((1, H)),
                full((H, H)),
                full((H, H)),
                full((1, H)),
                full((H, 1)),
            ],
            out_specs=[
                pl.BlockSpec((1, 1, K1), lambda b, c: (b, 0, 0)),
                pl.BlockSpec((1, 1, K1), lambda b, c: (b, 0, 0)),
                pl.BlockSpec((1, K1, H), lambda b, c: (b, 0, 0)),
                pl.BlockSpec((1, 1, 1), lambda b, c: (b, 0, 0)),
            ],
        ),
        out_shape=[
            jax.ShapeDtypeStruct((B, 1, K1), jnp.float32),
            jax.ShapeDtypeStruct((B, 1, K1), jnp.int32),
            jax.ShapeDtypeStruct((B, K1, H), jnp.float32),
            jax.ShapeDtypeStruct((B, 1, 1), jnp.float32),
        ],
        compiler_params=pltpu.CompilerParams(
            dimension_semantics=("arbitrary",),
        ),
    )(cls_i, hs_bf, pmr, W_start.T.astype(bf), b_start.reshape(1, 1),
      W_e0[H:].astype(bf), b_e0.reshape(1, H), W_a0[:H].astype(bf),
      W_a0[H:].astype(bf), b_a0.reshape(1, H), W_a1.astype(bf))

    el = pl.pallas_call(
        _end_logits_body,
        grid=(B, NT),
        in_specs=[
            pl.BlockSpec((1, TILE, H), lambda b, t: (b, t, 0)),
            pl.BlockSpec((1, TILE, 1), lambda b, t: (b, t, 0)),
            pl.BlockSpec((1, K1, H), lambda b, t: (b, 0, 0)),
            full((H, H)),
            full((1, H)),
            full((1, H)),
            full((H, 1)),
            full((1, 1)),
        ],
        out_specs=pl.BlockSpec((1, TILE, K1), lambda b, t: (b, t, 0)),
        out_shape=jax.ShapeDtypeStruct((B, S, K1), jnp.float32),
        compiler_params=pltpu.CompilerParams(
            dimension_semantics=("parallel", "arbitrary"),
        ),
    )(hs_bf, pm3, off, W_e0[:H].astype(bf), ln_g.reshape(1, H),
      ln_b.reshape(1, H), W_e1.astype(bf), b_e1.reshape(1, 1))

    el3 = jnp.transpose(el, (0, 2, 1))
    etv, eti = pl.pallas_call(
        _end_topk_body,
        grid=(B,),
        in_specs=[pl.BlockSpec((1, K1, S), lambda b: (b, 0, 0))],
        out_specs=[
            pl.BlockSpec((1, K1, K2), lambda b: (b, 0, 0)),
            pl.BlockSpec((1, K1, K2), lambda b: (b, 0, 0)),
        ],
        out_shape=[
            jax.ShapeDtypeStruct((B, K1, K2), jnp.float32),
            jax.ShapeDtypeStruct((B, K1, K2), jnp.int32),
        ],
        compiler_params=pltpu.CompilerParams(
            dimension_semantics=("arbitrary",),
        ),
    )(el3)

    start_top_log_probs = stv[:, 0, :]
    start_top_index = sti[:, 0, :]
    end_top_log_probs = jnp.transpose(etv, (0, 2, 1)).reshape(B, K1 * K2)
    end_top_index = jnp.transpose(eti, (0, 2, 1)).reshape(B, K1 * K2)
    cls_logits = clsl.reshape(B)
    return (start_top_log_probs, start_top_index, end_top_log_probs,
            end_top_index, cls_logits)


# speedup vs baseline: 2.7718x; 1.2075x over previous
import jax
import jax.numpy as jnp
from jax.experimental import pallas as pl
from jax.experimental.pallas import tpu as pltpu

_EPS = 1e-12
_NEG = -1e30


def _bdot(a, b):
    return jax.lax.dot_general(a, b, (((1,), (0,)), ((), ())),
                               preferred_element_type=jnp.float32)


def _topk_rows(scores, iota, kk, kio, s_len):
    vvec = jnp.zeros((1, kk), jnp.float32)
    ivec = jnp.zeros((1, kk), jnp.int32)
    cur = scores
    for k in range(kk):
        mk = jnp.max(cur)
        ik = jnp.min(jnp.where(cur == mk, iota, s_len))
        vvec = jnp.where(kio == k, mk, vvec)
        ivec = jnp.where(kio == k, ik, ivec)
        cur = jnp.where(iota == ik, _NEG, cur)
    return vvec, ivec


def _start_body(cls_ref, hs_ref, pm_ref, wst_ref, bst_ref, we0b_ref, be0_ref,
                wa0t_ref, wa0b_ref, ba0_ref, wa1_ref,
                stv_ref, sti_ref, off_ref, clsl_ref, hsb_ref):
    b = pl.program_id(0)
    S, H = hs_ref.shape[1], hs_ref.shape[2]
    K1 = stv_ref.shape[2]
    hs = hs_ref[0].astype(jnp.bfloat16)
    hsb_ref[0] = hs
    pm = pm_ref[0]
    iota = jax.lax.broadcasted_iota(jnp.int32, (1, S), 1)
    kio = jax.lax.broadcasted_iota(jnp.int32, (1, K1), 1)

    sl = jax.lax.dot_general(wst_ref[...], hs, (((1,), (1,)), ((), ())),
                             preferred_element_type=jnp.float32)
    sl = (sl + bst_ref[0, 0]) * (1.0 - pm) - 1e30 * pm
    m0 = jnp.max(sl)
    e0 = jnp.exp(sl - m0)
    d0 = jnp.sum(e0)

    lvec, ivec = _topk_rows(sl, iota, K1, kio, S)
    stv_ref[0] = jnp.exp(lvec - m0) / d0
    sti_ref[0] = ivec

    def _gather_row(ix):
        return hs_ref[0, pl.ds(ix, 1), :]

    rows = []
    for k in range(K1):
        ik = jnp.min(jnp.where(kio == k, ivec, S))
        rows.append(_gather_row(ik))
    ss = jnp.concatenate(rows, axis=0).astype(jnp.bfloat16)
    off_ref[0] = _bdot(ss, we0b_ref[...]) + be0_ref[...]

    agg = _bdot(e0.astype(jnp.bfloat16), hs)
    agg = agg * (1.0 / d0)
    cb = cls_ref[b]
    ctok = _gather_row(cb).astype(jnp.bfloat16)
    xa = jnp.tanh(_bdot(agg.astype(jnp.bfloat16), wa0t_ref[...])
                  + _bdot(ctok, wa0b_ref[...]) + ba0_ref[...])
    clsl_ref[0] = _bdot(xa.astype(jnp.bfloat16), wa1_ref[...])


def _end_logits_body(hs_ref, pm_ref, off_ref, we0t_ref, lng_ref, lnb_ref,
                     we1_ref, be1_ref, el_ref):
    T, H = hs_ref.shape[1], hs_ref.shape[2]
    K1 = off_ref.shape[1]
    hs = hs_ref[0]
    pm = pm_ref[0]
    keep = 1.0 - pm

    base = _bdot(hs, we0t_ref[...])
    lng = lng_ref[...]
    lnb = lnb_ref[...]
    inv_h = 1.0 / H

    cols = []
    for k in range(K1):
        xk = jnp.tanh(base + off_ref[0, k:k + 1, :])
        mu = jnp.sum(xk, axis=1, keepdims=True) * inv_h
        msq = jnp.sum(xk * xk, axis=1, keepdims=True) * inv_h
        rstd = 1.0 / jnp.sqrt(msq - mu * mu + _EPS)
        xn = (xk - mu) * rstd * lng + lnb
        el = _bdot(xn.astype(jnp.bfloat16), we1_ref[...]) + be1_ref[0, 0]
        cols.append(el * keep - 1e30 * pm)
    el_ref[0] = jnp.concatenate(cols, axis=1)


def _end_topk_body(el_ref, etv_ref, eti_ref):
    S = el_ref.shape[2]
    K1 = etv_ref.shape[1]
    K2 = etv_ref.shape[2]
    el = el_ref[0]
    iota = jax.lax.broadcasted_iota(jnp.int32, (K1, S), 1)
    me = jnp.max(el, axis=1, keepdims=True)
    de = jnp.sum(jnp.exp(el - me), axis=1, keepdims=True)
    cur = el
    vcols = []
    icols = []
    for k2 in range(K2):
        mk = jnp.max(cur, axis=1, keepdims=True)
        ik = jnp.min(jnp.where(cur == mk, iota, S), axis=1, keepdims=True)
        vcols.append(jnp.exp(mk - me) / de)
        icols.append(ik)
        cur = jnp.where(iota == ik, _NEG, cur)
    etv_ref[0] = jnp.concatenate(vcols, axis=1)
    eti_ref[0] = jnp.concatenate(icols, axis=1)


def kernel(hidden_states, p_mask, cls_index, W_start, b_start, W_e0, b_e0,
           ln_g, ln_b, W_e1, b_e1, W_a0, b_a0, W_a1):
    B, S, H = hidden_states.shape
    K1, K2 = 5, 5
    TILE = 1024
    NT = S // TILE

    bf = jnp.bfloat16
    cls_i = cls_index.astype(jnp.int32)
    pm3 = p_mask[:, :, None]
    pmr = p_mask[:, None, :]

    full = lambda shape: pl.BlockSpec(shape, lambda *a: (0,) * len(shape))

    stv, sti, off, clsl, hs_bf = pl.pallas_call(
        _start_body,
        grid_spec=pltpu.PrefetchScalarGridSpec(
            num_scalar_prefetch=1,
            grid=(B,),
            in_specs=[
                pl.BlockSpec((1, S, H), lambda b, c: (b, 0, 0)),
                pl.BlockSpec((1, 1, S), lambda b, c: (b, 0, 0)),
                full((1, H)),
                full((1, 1)),
                full((H, H)),
                full((1, H)),
                full((H, H)),
                full((H, H)),
                full((1, H)),
                full((H, 1)),
            ],
            out_specs=[
                pl.BlockSpec((1, 1, K1), lambda b, c: (b, 0, 0)),
                pl.BlockSpec((1, 1, K1), lambda b, c: (b, 0, 0)),
                pl.BlockSpec((1, K1, H), lambda b, c: (b, 0, 0)),
                pl.BlockSpec((1, 1, 1), lambda b, c: (b, 0, 0)),
                pl.BlockSpec((1, S, H), lambda b, c: (b, 0, 0)),
            ],
        ),
        out_shape=[
            jax.ShapeDtypeStruct((B, 1, K1), jnp.float32),
            jax.ShapeDtypeStruct((B, 1, K1), jnp.int32),
            jax.ShapeDtypeStruct((B, K1, H), jnp.float32),
            jax.ShapeDtypeStruct((B, 1, 1), jnp.float32),
            jax.ShapeDtypeStruct((B, S, H), bf),
        ],
        compiler_params=pltpu.CompilerParams(
            dimension_semantics=("arbitrary",),
        ),
    )(cls_i, hidden_states, pmr, W_start.T.astype(bf), b_start.reshape(1, 1),
      W_e0[H:].astype(bf), b_e0.reshape(1, H), W_a0[:H].astype(bf),
      W_a0[H:].astype(bf), b_a0.reshape(1, H), W_a1.astype(bf))

    el = pl.pallas_call(
        _end_logits_body,
        grid=(B, NT),
        in_specs=[
            pl.BlockSpec((1, TILE, H), lambda b, t: (b, t, 0)),
            pl.BlockSpec((1, TILE, 1), lambda b, t: (b, t, 0)),
            pl.BlockSpec((1, K1, H), lambda b, t: (b, 0, 0)),
            full((H, H)),
            full((1, H)),
            full((1, H)),
            full((H, 1)),
            full((1, 1)),
        ],
        out_specs=pl.BlockSpec((1, TILE, K1), lambda b, t: (b, t, 0)),
        out_shape=jax.ShapeDtypeStruct((B, S, K1), jnp.float32),
        compiler_params=pltpu.CompilerParams(
            dimension_semantics=("parallel", "arbitrary"),
        ),
    )(hs_bf, pm3, off, W_e0[:H].astype(bf), ln_g.reshape(1, H),
      ln_b.reshape(1, H), W_e1.astype(bf), b_e1.reshape(1, 1))

    el3 = jnp.transpose(el, (0, 2, 1))
    etv, eti = pl.pallas_call(
        _end_topk_body,
        grid=(B,),
        in_specs=[pl.BlockSpec((1, K1, S), lambda b: (b, 0, 0))],
        out_specs=[
            pl.BlockSpec((1, K1, K2), lambda b: (b, 0, 0)),
            pl.BlockSpec((1, K1, K2), lambda b: (b, 0, 0)),
        ],
        out_shape=[
            jax.ShapeDtypeStruct((B, K1, K2), jnp.float32),
            jax.ShapeDtypeStruct((B, K1, K2), jnp.int32),
        ],
        compiler_params=pltpu.CompilerParams(
            dimension_semantics=("arbitrary",),
        ),
    )(el3)

    start_top_log_probs = stv[:, 0, :]
    start_top_index = sti[:, 0, :]
    end_top_log_probs = jnp.transpose(etv, (0, 2, 1)).reshape(B, K1 * K2)
    end_top_index = jnp.transpose(eti, (0, 2, 1)).reshape(B, K1 * K2)
    cls_logits = clsl.reshape(B)
    return (start_top_log_probs, start_top_index, end_top_log_probs,
            end_top_index, cls_logits)
